# baseline (speedup 1.0000x reference)
import functools

import jax
import jax.numpy as jnp
import numpy as np
from jax.experimental import pallas as pl
from jax.experimental.pallas import tpu as pltpu

_INTERPRET = False


def _r2(v):
    return v.reshape(1, -1)


def _lin_bn_relu_body(x_ref, w_ref, b_ref, g_ref, bt_ref, o_ref):
    x = x_ref[...]
    y = jnp.dot(x, w_ref[...], preferred_element_type=jnp.float32) + b_ref[...]
    mu = jnp.mean(y, axis=0, keepdims=True)
    yc = y - mu
    var = jnp.mean(yc * yc, axis=0, keepdims=True)
    o_ref[...] = jnp.maximum(
        g_ref[...] * yc * jax.lax.rsqrt(var + 1e-5) + bt_ref[...], 0.0)


def _lin_bn_relu(p, x):
    b, n, din = x.shape
    dout = p["W"].shape[1]
    out = pl.pallas_call(
        _lin_bn_relu_body,
        out_shape=jax.ShapeDtypeStruct((b * n, dout), jnp.float32),
        interpret=_INTERPRET,
    )(x.reshape(b * n, din), p["W"], _r2(p["b"]), _r2(p["g"]), _r2(p["beta"]))
    return out.reshape(b, n, dout)


def _auxcross_body(x_ref, aux_ref,
                   w1, b1, g1, t1, w2, b2, g2, t2, wh, bh,
                   wq, bq, wk, bk, wv, bv, wo, bo, lg, lb, o_ref):
    def bn(y, g, t):
        mu = jnp.mean(y, axis=0, keepdims=True)
        yc = y - mu
        var = jnp.mean(yc * yc, axis=0, keepdims=True)
        return jnp.maximum(g[...] * yc * jax.lax.rsqrt(var + 1e-5) + t[...], 0.0)

    aux = aux_ref[...]
    h = bn(jnp.dot(aux, w1[...], preferred_element_type=jnp.float32) + b1[...], g1, t1)
    h = bn(jnp.dot(h, w2[...], preferred_element_type=jnp.float32) + b2[...], g2, t2)
    tok = jnp.dot(h, wh[...], preferred_element_type=jnp.float32) + bh[...]

    x = x_ref[...]
    d = x.shape[1]
    q = jnp.dot(x, wq[...], preferred_element_type=jnp.float32) + bq[...]
    k = jnp.dot(tok, wk[...], preferred_element_type=jnp.float32) + bk[...]
    v = jnp.dot(tok, wv[...], preferred_element_type=jnp.float32) + bv[...]
    gate = jax.nn.sigmoid(jnp.sum(q * k, axis=1, keepdims=True) * (d ** -0.5))
    h2 = x + jnp.dot(v * gate, wo[...], preferred_element_type=jnp.float32) + bo[...]
    mu = jnp.mean(h2, axis=1, keepdims=True)
    hc = h2 - mu
    var = jnp.mean(hc * hc, axis=1, keepdims=True)
    o_ref[...] = lg[...] * hc * jax.lax.rsqrt(var + 1e-5) + lb[...]


def _auxcross(params, stage, x, aux):
    b, n, d = x.shape
    t0, t1p = params["aux_trunk"]
    hd = params["aux_heads"][stage]
    c = params["cross"][stage]
    args = [x.reshape(b * n, d), aux.reshape(b * n, aux.shape[-1]),
            t0["W"], _r2(t0["b"]), _r2(t0["g"]), _r2(t0["beta"]),
            t1p["W"], _r2(t1p["b"]), _r2(t1p["g"]), _r2(t1p["beta"]),
            hd["W"], _r2(hd["b"]),
            c["wq"]["W"], _r2(c["wq"]["b"]), c["wk"]["W"], _r2(c["wk"]["b"]),
            c["wv"]["W"], _r2(c["wv"]["b"]), c["out"]["W"], _r2(c["out"]["b"]),
            _r2(c["ln_g"]), _r2(c["ln_b"])]
    out = pl.pallas_call(
        _auxcross_body,
        out_shape=jax.ShapeDtypeStruct((b * n, d), jnp.float32),
        interpret=_INTERPRET,
    )(*args)
    return out.reshape(b, n, d)


def _fps_body(px_ref, py_ref, pz_ref, cx_ref, cy_ref, cz_ref, *, m):
    px = px_ref[...]
    py = py_ref[...]
    pz = pz_ref[...]
    bsz, n = px.shape
    d = ((px - px[:, 0:1]) ** 2 + (py - py[:, 0:1]) ** 2
         + (pz - pz[:, 0:1]) ** 2)
    iota = jax.lax.broadcasted_iota(jnp.int32, (bsz, n), 1)
    iota_m = jax.lax.broadcasted_iota(jnp.int32, (bsz, m), 1)
    zm = jnp.zeros((bsz, m), jnp.float32)
    sel0 = (iota_m == 0).astype(jnp.float32)
    cx = sel0 * px[:, 0:1]
    cy = sel0 * py[:, 0:1]
    cz = sel0 * pz[:, 0:1]

    def body(t, carry):
        d, cx, cy, cz = carry
        mx = jnp.max(d, axis=1, keepdims=True)
        cand = jnp.where(d == mx, iota, n)
        i = jnp.min(cand, axis=1, keepdims=True)
        oh = iota == i
        pxi = jnp.sum(jnp.where(oh, px, 0.0), axis=1, keepdims=True)
        pyi = jnp.sum(jnp.where(oh, py, 0.0), axis=1, keepdims=True)
        pzi = jnp.sum(jnp.where(oh, pz, 0.0), axis=1, keepdims=True)
        sel = (iota_m == t).astype(jnp.float32)
        cx = cx + sel * pxi
        cy = cy + sel * pyi
        cz = cz + sel * pzi
        dn = (px - pxi) ** 2 + (py - pyi) ** 2 + (pz - pzi) ** 2
        return jnp.minimum(d, dn), cx, cy, cz

    _, cx, cy, cz = jax.lax.fori_loop(1, m, body, (d, cx, cy, cz))
    cx_ref[...] = cx
    cy_ref[...] = cy
    cz_ref[...] = cz


def _fps(pos, m):
    bsz, n, _ = pos.shape
    sds = jax.ShapeDtypeStruct((bsz, m), jnp.float32)
    cx, cy, cz = pl.pallas_call(
        functools.partial(_fps_body, m=m),
        out_shape=[sds, sds, sds],
        interpret=_INTERPRET,
    )(pos[:, :, 0], pos[:, :, 1], pos[:, :, 2])
    return jnp.stack([cx, cy, cz], axis=-1)


def _extract_nearest(d2, iota_c, n):
    mn = jnp.min(d2, axis=1, keepdims=True)
    cand = jnp.where(d2 == mn, iota_c, n)
    j = jnp.min(cand, axis=1, keepdims=True)
    ohb = iota_c == j
    return ohb, jnp.where(ohb, 1e30, d2)


def _split_hi_lo(s):
    hi = s.astype(jnp.bfloat16).astype(jnp.float32)
    return hi, s - hi


def _onehot_gather(ohb, s_hi, s_lo):
    ohf = ohb.astype(jnp.float32)
    return (jnp.dot(ohf, s_hi, preferred_element_type=jnp.float32)
            + jnp.dot(ohf, s_lo, preferred_element_type=jnp.float32))


def _pairwise_d2(a, bT):
    d2 = None
    for c in range(3):
        dd = a[:, c:c + 1] - bT[c:c + 1, :]
        d2 = dd * dd if d2 is None else d2 + dd * dd
    return d2


def _trans_body(xtr_ref, aux_ref, ctr_ref, posT_ref, xo_ref, ao_ref, *, k):
    xtr = xtr_ref[0]
    aux = aux_ref[0]
    ctr = ctr_ref[0]
    posT = posT_ref[0]
    n, dout = xtr.shape
    m = ctr.shape[0]
    d2 = _pairwise_d2(ctr, posT)
    iota_c = jax.lax.broadcasted_iota(jnp.int32, (m, n), 1)
    s_hi, s_lo = _split_hi_lo(jnp.concatenate([xtr, aux], axis=1))
    xmax = None
    asum = None
    for t in range(k):
        ohb, d2 = _extract_nearest(d2, iota_c, n)
        g = _onehot_gather(ohb, s_hi, s_lo)
        gx = g[:, :dout]
        ga = g[:, dout:]
        xmax = gx if xmax is None else jnp.maximum(xmax, gx)
        asum = ga if asum is None else asum + ga
    xo_ref[0] = xmax
    ao_ref[0] = asum / k


def _trans(xtr, aux, pos, centers, k):
    b, n, dout = xtr.shape
    m = centers.shape[1]
    posT = jnp.swapaxes(pos, 1, 2)

    def bat(*shape):
        return pl.BlockSpec((1,) + shape, lambda i, s=len(shape): (i,) + (0,) * s)

    xo, ao = pl.pallas_call(
        functools.partial(_trans_body, k=k),
        grid=(b,),
        compiler_params=pltpu.CompilerParams(dimension_semantics=("parallel",)),
        in_specs=[bat(n, dout), bat(n, 4), bat(m, 3), bat(3, n)],
        out_specs=[bat(m, dout), bat(m, 4)],
        out_shape=[jax.ShapeDtypeStruct((b, m, dout), jnp.float32),
                   jax.ShapeDtypeStruct((b, m, 4), jnp.float32)],
        interpret=_INTERPRET,
    )(xtr, aux, centers, posT)
    return xo, ao


def _tblock_body(x_ref, pos_ref, posT_ref,
                 wi, bi, wl, ws, wd, wp1, bp1, wp2, bp2,
                 wa1, ba1, wa2, ba2, wo, bo, o_ref, *, k):
    def lin_relu(a, w, bb):
        return jnp.maximum(
            jnp.dot(a, w[...], preferred_element_type=jnp.float32) + bb[...], 0.0)

    x = x_ref[0]
    pos = pos_ref[0]
    posT = posT_ref[0]
    n, d = x.shape
    x1 = lin_relu(x, wi, bi)
    xl = jnp.dot(x1, wl[...], preferred_element_type=jnp.float32)
    asrc = jnp.dot(x1, ws[...], preferred_element_type=jnp.float32)
    adst = jnp.dot(x1, wd[...], preferred_element_type=jnp.float32)

    d2 = _pairwise_d2(pos, posT)
    iota_r = jax.lax.broadcasted_iota(jnp.int32, (n, n), 0)
    iota_c = jax.lax.broadcasted_iota(jnp.int32, (n, n), 1)
    d2 = d2 + jnp.where(iota_r == iota_c, 1e10, 0.0)

    s_hi, s_lo = _split_hi_lo(
        jnp.concatenate([pos, asrc, xl], axis=1))
    chunk = 1 if n % 8 else max(1, min(k, 8192 // n))
    es = []
    vs = []
    for t0 in range(0, k, chunk):
        pds, als, xjs = [], [], []
        for t in range(t0, min(t0 + chunk, k)):
            ohb, d2 = _extract_nearest(d2, iota_c, n)
            g = _onehot_gather(ohb, s_hi, s_lo)
            pds.append(pos - g[:, :3])
            als.append(adst - g[:, 3:3 + d])
            xjs.append(g[:, 3 + d:])
        c = len(pds)
        pd_c = pds[0] if c == 1 else jnp.concatenate(pds, axis=0)
        al_c = als[0] if c == 1 else jnp.concatenate(als, axis=0)
        delta_c = lin_relu(lin_relu(pd_c, wp1, bp1), wp2, bp2)
        e_c = lin_relu(lin_relu(al_c + delta_c, wa1, ba1), wa2, ba2)
        for ci in range(c):
            sl = slice(ci * n, (ci + 1) * n)
            es.append(e_c[sl])
            vs.append(xjs[ci] + delta_c[sl])
    mx = functools.reduce(jnp.maximum, es)
    ssum = None
    acc = None
    for e, v in zip(es, vs):
        w = jnp.exp(e - mx)
        ssum = w if ssum is None else ssum + w
        acc = w * v if acc is None else acc + w * v
    o_ref[0] = lin_relu(acc / ssum, wo, bo)


def _tblock(p, x, pos, k):
    b, n, d = x.shape
    posT = jnp.swapaxes(pos, 1, 2)
    pars = [p["lin_in"]["W"], _r2(p["lin_in"]["b"]), p["lin"]["W"],
            p["lin_src"]["W"], p["lin_dst"]["W"],
            p["pos_nn"][0]["W"], _r2(p["pos_nn"][0]["b"]),
            p["pos_nn"][1]["W"], _r2(p["pos_nn"][1]["b"]),
            p["attn_nn"][0]["W"], _r2(p["attn_nn"][0]["b"]),
            p["attn_nn"][1]["W"], _r2(p["attn_nn"][1]["b"]),
            p["lin_out"]["W"], _r2(p["lin_out"]["b"])]

    def bat(*shape):
        return pl.BlockSpec((1,) + shape, lambda i, s=len(shape): (i,) + (0,) * s)

    def shared(a):
        return pl.BlockSpec(a.shape, lambda i, s=a.ndim: (0,) * s)

    out = pl.pallas_call(
        functools.partial(_tblock_body, k=k),
        grid=(b,),
        compiler_params=pltpu.CompilerParams(dimension_semantics=("parallel",)),
        in_specs=[bat(n, d), bat(n, 3), bat(3, n)] + [shared(a) for a in pars],
        out_specs=bat(n, d),
        out_shape=jax.ShapeDtypeStruct((b, n, d), jnp.float32),
        interpret=_INTERPRET,
    )(x, pos, posT, *pars)
    return out


def _head_body(x_ref, ac_ref, w1, b1, w2, b2, o_ref, *, bsz, n):
    pooled = []
    for b in range(bsz):
        xb = x_ref[pl.ds(b * n, n), :]
        w = jax.nn.sigmoid(ac_ref[b:b + 1, :])
        s = jnp.dot(w, xb, precision=jax.lax.Precision.HIGHEST,
                    preferred_element_type=jnp.float32)
        ws = jnp.clip(jnp.sum(w, axis=1, keepdims=True), 1e-6)
        pooled.append(s / ws)
    pooled = jnp.concatenate(pooled, axis=0)
    h = jnp.maximum(
        jnp.dot(pooled, w1[...], preferred_element_type=jnp.float32) + b1[...], 0.0)
    o_ref[...] = jnp.dot(h, w2[...], preferred_element_type=jnp.float32) + b2[...]


def _head(p, x, aux):
    b, n, d = x.shape
    return pl.pallas_call(
        functools.partial(_head_body, bsz=b, n=n),
        out_shape=jax.ShapeDtypeStruct((b, p[1]["W"].shape[1]), jnp.float32),
        interpret=_INTERPRET,
    )(x.reshape(b * n, d), aux[:, :, 2], p[0]["W"], _r2(p[0]["b"]),
      p[1]["W"], _r2(p[1]["b"]))


def kernel(data, params):
    pos = data[..., :3]
    aux = data[..., 3:7]
    n = pos.shape[1]

    x = _lin_bn_relu(params["mlp_input"], pos)
    x = _tblock(params["tr_in"], x, pos, min(16, n))
    x = _auxcross(params, 0, x, aux)
    for i in range(4):
        m = int(np.ceil(n * 0.25))
        centers = _fps(pos, m)
        xtr = _lin_bn_relu(params["td"][i], x)
        x, aux = _trans(xtr, aux, pos, centers, min(16, n))
        pos = centers
        n = m
        x = _tblock(params["tr_down"][i], x, pos, min(16, n))
        x = _auxcross(params, i + 1, x, aux)
    return _head(params["head"], x, aux)


# speedup vs baseline: 1.1570x; 1.1570x over previous
import functools

import jax
import jax.numpy as jnp
import numpy as np
from jax.experimental import pallas as pl
from jax.experimental.pallas import tpu as pltpu

_INTERPRET = False


def _r2(v):
    return v.reshape(1, -1)


def _lin_bn_relu_body(x_ref, w_ref, b_ref, g_ref, bt_ref, o_ref):
    x = x_ref[...]
    y = jnp.dot(x, w_ref[...], preferred_element_type=jnp.float32) + b_ref[...]
    mu = jnp.mean(y, axis=0, keepdims=True)
    yc = y - mu
    var = jnp.mean(yc * yc, axis=0, keepdims=True)
    o_ref[...] = jnp.maximum(
        g_ref[...] * yc * jax.lax.rsqrt(var + 1e-5) + bt_ref[...], 0.0)


def _lin_bn_relu(p, x):
    b, n, din = x.shape
    dout = p["W"].shape[1]
    out = pl.pallas_call(
        _lin_bn_relu_body,
        out_shape=jax.ShapeDtypeStruct((b * n, dout), jnp.float32),
        interpret=_INTERPRET,
    )(x.reshape(b * n, din), p["W"], _r2(p["b"]), _r2(p["g"]), _r2(p["beta"]))
    return out.reshape(b, n, dout)


def _auxcross_body(x_ref, aux_ref,
                   w1, b1, g1, t1, w2, b2, g2, t2, wh, bh,
                   wq, bq, wk, bk, wv, bv, wo, bo, lg, lb, o_ref):
    def bn(y, g, t):
        mu = jnp.mean(y, axis=0, keepdims=True)
        yc = y - mu
        var = jnp.mean(yc * yc, axis=0, keepdims=True)
        return jnp.maximum(g[...] * yc * jax.lax.rsqrt(var + 1e-5) + t[...], 0.0)

    aux = aux_ref[...]
    h = bn(jnp.dot(aux, w1[...], preferred_element_type=jnp.float32) + b1[...], g1, t1)
    h = bn(jnp.dot(h, w2[...], preferred_element_type=jnp.float32) + b2[...], g2, t2)
    tok = jnp.dot(h, wh[...], preferred_element_type=jnp.float32) + bh[...]

    x = x_ref[...]
    d = x.shape[1]
    q = jnp.dot(x, wq[...], preferred_element_type=jnp.float32) + bq[...]
    k = jnp.dot(tok, wk[...], preferred_element_type=jnp.float32) + bk[...]
    v = jnp.dot(tok, wv[...], preferred_element_type=jnp.float32) + bv[...]
    gate = jax.nn.sigmoid(jnp.sum(q * k, axis=1, keepdims=True) * (d ** -0.5))
    h2 = x + jnp.dot(v * gate, wo[...], preferred_element_type=jnp.float32) + bo[...]
    mu = jnp.mean(h2, axis=1, keepdims=True)
    hc = h2 - mu
    var = jnp.mean(hc * hc, axis=1, keepdims=True)
    o_ref[...] = lg[...] * hc * jax.lax.rsqrt(var + 1e-5) + lb[...]


def _auxcross(params, stage, x, aux):
    b, n, d = x.shape
    t0, t1p = params["aux_trunk"]
    hd = params["aux_heads"][stage]
    c = params["cross"][stage]
    args = [x.reshape(b * n, d), aux.reshape(b * n, aux.shape[-1]),
            t0["W"], _r2(t0["b"]), _r2(t0["g"]), _r2(t0["beta"]),
            t1p["W"], _r2(t1p["b"]), _r2(t1p["g"]), _r2(t1p["beta"]),
            hd["W"], _r2(hd["b"]),
            c["wq"]["W"], _r2(c["wq"]["b"]), c["wk"]["W"], _r2(c["wk"]["b"]),
            c["wv"]["W"], _r2(c["wv"]["b"]), c["out"]["W"], _r2(c["out"]["b"]),
            _r2(c["ln_g"]), _r2(c["ln_b"])]
    out = pl.pallas_call(
        _auxcross_body,
        out_shape=jax.ShapeDtypeStruct((b * n, d), jnp.float32),
        interpret=_INTERPRET,
    )(*args)
    return out.reshape(b, n, d)


def _fps_body(px_ref, py_ref, pz_ref, cx_ref, cy_ref, cz_ref, *, m):
    px = px_ref[...]
    py = py_ref[...]
    pz = pz_ref[...]
    bsz, n = px.shape
    d = ((px - px[:, 0:1]) ** 2 + (py - py[:, 0:1]) ** 2
         + (pz - pz[:, 0:1]) ** 2)
    iota = jax.lax.broadcasted_iota(jnp.int32, (bsz, n), 1)
    iota_m = jax.lax.broadcasted_iota(jnp.int32, (bsz, m), 1)
    zm = jnp.zeros((bsz, m), jnp.float32)
    sel0 = (iota_m == 0).astype(jnp.float32)
    cx = sel0 * px[:, 0:1]
    cy = sel0 * py[:, 0:1]
    cz = sel0 * pz[:, 0:1]

    def body(t, carry):
        d, cx, cy, cz = carry
        mx = jnp.max(d, axis=1, keepdims=True)
        cand = jnp.where(d == mx, iota, n)
        i = jnp.min(cand, axis=1, keepdims=True)
        oh = iota == i
        pxi = jnp.sum(jnp.where(oh, px, 0.0), axis=1, keepdims=True)
        pyi = jnp.sum(jnp.where(oh, py, 0.0), axis=1, keepdims=True)
        pzi = jnp.sum(jnp.where(oh, pz, 0.0), axis=1, keepdims=True)
        sel = (iota_m == t).astype(jnp.float32)
        cx = cx + sel * pxi
        cy = cy + sel * pyi
        cz = cz + sel * pzi
        dn = (px - pxi) ** 2 + (py - pyi) ** 2 + (pz - pzi) ** 2
        return jnp.minimum(d, dn), cx, cy, cz

    _, cx, cy, cz = jax.lax.fori_loop(1, m, body, (d, cx, cy, cz))
    cx_ref[...] = cx
    cy_ref[...] = cy
    cz_ref[...] = cz


def _fps(pos, m):
    bsz, n, _ = pos.shape
    sds = jax.ShapeDtypeStruct((bsz, m), jnp.float32)
    cx, cy, cz = pl.pallas_call(
        functools.partial(_fps_body, m=m),
        out_shape=[sds, sds, sds],
        interpret=_INTERPRET,
    )(pos[:, :, 0], pos[:, :, 1], pos[:, :, 2])
    return jnp.stack([cx, cy, cz], axis=-1)


def _extract_nearest(d2, iota_c, n):
    mn = jnp.min(d2, axis=1, keepdims=True)
    cand = jnp.where(d2 == mn, iota_c, n)
    j = jnp.min(cand, axis=1, keepdims=True)
    ohb = iota_c == j
    return ohb, jnp.where(ohb, 1e30, d2)


def _split_hi_lo(s):
    n, w = s.shape
    wp = -(-w // 128) * 128
    if wp > w:
        s = jnp.concatenate([s, jnp.zeros((n, wp - w), jnp.float32)], axis=1)
    hi = s.astype(jnp.bfloat16).astype(jnp.float32)
    return jnp.concatenate([hi, s - hi], axis=1), wp


def _onehot_gather(ohb, s2, wp):
    g2 = jnp.dot(ohb.astype(jnp.float32), s2,
                 preferred_element_type=jnp.float32)
    return g2[:, :wp] + g2[:, wp:]


def _pairwise_d2(a, bT):
    d2 = None
    for c in range(3):
        dd = a[:, c:c + 1] - bT[c:c + 1, :]
        d2 = dd * dd if d2 is None else d2 + dd * dd
    return d2


def _trans_body(xtr_ref, aux_ref, ctr_ref, posT_ref, xo_ref, ao_ref, *, k):
    xtr = xtr_ref[0]
    aux = aux_ref[0]
    ctr = ctr_ref[0]
    posT = posT_ref[0]
    n, dout = xtr.shape
    m = ctr.shape[0]
    d2 = _pairwise_d2(ctr, posT)
    iota_c = jax.lax.broadcasted_iota(jnp.int32, (m, n), 1)
    s2, wp = _split_hi_lo(jnp.concatenate([xtr, aux], axis=1))
    xmax = None
    asum = None
    for t in range(k):
        ohb, d2 = _extract_nearest(d2, iota_c, n)
        g = _onehot_gather(ohb, s2, wp)
        gx = g[:, :dout]
        ga = g[:, dout:dout + 4]
        xmax = gx if xmax is None else jnp.maximum(xmax, gx)
        asum = ga if asum is None else asum + ga
    xo_ref[0] = xmax
    ao_ref[0] = asum / k


def _trans(xtr, aux, pos, centers, k):
    b, n, dout = xtr.shape
    m = centers.shape[1]
    posT = jnp.swapaxes(pos, 1, 2)

    def bat(*shape):
        return pl.BlockSpec((1,) + shape, lambda i, s=len(shape): (i,) + (0,) * s)

    xo, ao = pl.pallas_call(
        functools.partial(_trans_body, k=k),
        grid=(b,),
        compiler_params=pltpu.CompilerParams(dimension_semantics=("parallel",)),
        in_specs=[bat(n, dout), bat(n, 4), bat(m, 3), bat(3, n)],
        out_specs=[bat(m, dout), bat(m, 4)],
        out_shape=[jax.ShapeDtypeStruct((b, m, dout), jnp.float32),
                   jax.ShapeDtypeStruct((b, m, 4), jnp.float32)],
        interpret=_INTERPRET,
    )(xtr, aux, centers, posT)
    return xo, ao


def _tblock_body(x_ref, pos_ref, posT_ref,
                 wi, bi, wl, ws, wd, wp1, bp1, wp2, bp2,
                 wa1, ba1, wa2, ba2, wo, bo, o_ref, *, k):
    def lin_relu(a, w, bb):
        return jnp.maximum(
            jnp.dot(a, w[...], preferred_element_type=jnp.float32) + bb[...], 0.0)

    x = x_ref[0]
    pos = pos_ref[0]
    posT = posT_ref[0]
    n, d = x.shape
    x1 = lin_relu(x, wi, bi)
    xl = jnp.dot(x1, wl[...], preferred_element_type=jnp.float32)
    asrc = jnp.dot(x1, ws[...], preferred_element_type=jnp.float32)
    adst = jnp.dot(x1, wd[...], preferred_element_type=jnp.float32)

    d2 = _pairwise_d2(pos, posT)
    iota_r = jax.lax.broadcasted_iota(jnp.int32, (n, n), 0)
    iota_c = jax.lax.broadcasted_iota(jnp.int32, (n, n), 1)
    d2 = d2 + jnp.where(iota_r == iota_c, 1e10, 0.0)

    s2, wp = _split_hi_lo(
        jnp.concatenate([pos, asrc, xl], axis=1))
    chunk = 1 if n % 8 else max(1, min(k, 4096 // n))
    es = []
    vs = []
    for t0 in range(0, k, chunk):
        pds, als, xjs = [], [], []
        for t in range(t0, min(t0 + chunk, k)):
            ohb, d2 = _extract_nearest(d2, iota_c, n)
            g = _onehot_gather(ohb, s2, wp)
            pds.append(pos - g[:, :3])
            als.append(adst - g[:, 3:3 + d])
            xjs.append(g[:, 3 + d:3 + 2 * d])
        c = len(pds)
        pd_c = pds[0] if c == 1 else jnp.concatenate(pds, axis=0)
        al_c = als[0] if c == 1 else jnp.concatenate(als, axis=0)
        delta_c = lin_relu(lin_relu(pd_c, wp1, bp1), wp2, bp2)
        e_c = lin_relu(lin_relu(al_c + delta_c, wa1, ba1), wa2, ba2)
        for ci in range(c):
            sl = slice(ci * n, (ci + 1) * n)
            es.append(e_c[sl])
            vs.append(xjs[ci] + delta_c[sl])
    mx = functools.reduce(jnp.maximum, es)
    ssum = None
    acc = None
    for e, v in zip(es, vs):
        w = jnp.exp(e - mx)
        ssum = w if ssum is None else ssum + w
        acc = w * v if acc is None else acc + w * v
    o_ref[0] = lin_relu(acc / ssum, wo, bo)


def _tblock(p, x, pos, k):
    b, n, d = x.shape
    posT = jnp.swapaxes(pos, 1, 2)
    pars = [p["lin_in"]["W"], _r2(p["lin_in"]["b"]), p["lin"]["W"],
            p["lin_src"]["W"], p["lin_dst"]["W"],
            p["pos_nn"][0]["W"], _r2(p["pos_nn"][0]["b"]),
            p["pos_nn"][1]["W"], _r2(p["pos_nn"][1]["b"]),
            p["attn_nn"][0]["W"], _r2(p["attn_nn"][0]["b"]),
            p["attn_nn"][1]["W"], _r2(p["attn_nn"][1]["b"]),
            p["lin_out"]["W"], _r2(p["lin_out"]["b"])]

    def bat(*shape):
        return pl.BlockSpec((1,) + shape, lambda i, s=len(shape): (i,) + (0,) * s)

    def shared(a):
        return pl.BlockSpec(a.shape, lambda i, s=a.ndim: (0,) * s)

    out = pl.pallas_call(
        functools.partial(_tblock_body, k=k),
        grid=(b,),
        compiler_params=pltpu.CompilerParams(dimension_semantics=("parallel",)),
        in_specs=[bat(n, d), bat(n, 3), bat(3, n)] + [shared(a) for a in pars],
        out_specs=bat(n, d),
        out_shape=jax.ShapeDtypeStruct((b, n, d), jnp.float32),
        interpret=_INTERPRET,
    )(x, pos, posT, *pars)
    return out


def _head_body(x_ref, ac_ref, w1, b1, w2, b2, o_ref, *, bsz, n):
    pooled = []
    for b in range(bsz):
        xb = x_ref[pl.ds(b * n, n), :]
        w = jax.nn.sigmoid(ac_ref[b:b + 1, :])
        s = jnp.dot(w, xb, precision=jax.lax.Precision.HIGHEST,
                    preferred_element_type=jnp.float32)
        ws = jnp.clip(jnp.sum(w, axis=1, keepdims=True), 1e-6)
        pooled.append(s / ws)
    pooled = jnp.concatenate(pooled, axis=0)
    h = jnp.maximum(
        jnp.dot(pooled, w1[...], preferred_element_type=jnp.float32) + b1[...], 0.0)
    o_ref[...] = jnp.dot(h, w2[...], preferred_element_type=jnp.float32) + b2[...]


def _head(p, x, aux):
    b, n, d = x.shape
    return pl.pallas_call(
        functools.partial(_head_body, bsz=b, n=n),
        out_shape=jax.ShapeDtypeStruct((b, p[1]["W"].shape[1]), jnp.float32),
        interpret=_INTERPRET,
    )(x.reshape(b * n, d), aux[:, :, 2], p[0]["W"], _r2(p[0]["b"]),
      p[1]["W"], _r2(p[1]["b"]))


def kernel(data, params):
    pos = data[..., :3]
    aux = data[..., 3:7]
    n = pos.shape[1]

    x = _lin_bn_relu(params["mlp_input"], pos)
    x = _tblock(params["tr_in"], x, pos, min(16, n))
    x = _auxcross(params, 0, x, aux)
    for i in range(4):
        m = int(np.ceil(n * 0.25))
        centers = _fps(pos, m)
        xtr = _lin_bn_relu(params["td"][i], x)
        x, aux = _trans(xtr, aux, pos, centers, min(16, n))
        pos = centers
        n = m
        x = _tblock(params["tr_down"][i], x, pos, min(16, n))
        x = _auxcross(params, i + 1, x, aux)
    return _head(params["head"], x, aux)


# speedup vs baseline: 1.1578x; 1.0007x over previous
import functools

import jax
import jax.numpy as jnp
import numpy as np
from jax.experimental import pallas as pl
from jax.experimental.pallas import tpu as pltpu


def _r2(v):
    return v.reshape(1, -1)


def _lin_bn_relu_body(x_ref, w_ref, b_ref, g_ref, bt_ref, o_ref):
    x = x_ref[...]
    y = jnp.dot(x, w_ref[...], preferred_element_type=jnp.float32) + b_ref[...]
    mu = jnp.mean(y, axis=0, keepdims=True)
    yc = y - mu
    var = jnp.mean(yc * yc, axis=0, keepdims=True)
    o_ref[...] = jnp.maximum(
        g_ref[...] * yc * jax.lax.rsqrt(var + 1e-5) + bt_ref[...], 0.0)


def _lin_bn_relu(p, x):
    b, n, din = x.shape
    dout = p["W"].shape[1]
    out = pl.pallas_call(
        _lin_bn_relu_body,
        out_shape=jax.ShapeDtypeStruct((b * n, dout), jnp.float32),
    )(x.reshape(b * n, din), p["W"], _r2(p["b"]), _r2(p["g"]), _r2(p["beta"]))
    return out.reshape(b, n, dout)


def _auxcross_body(x_ref, aux_ref,
                   w1, b1, g1, t1, w2, b2, g2, t2, wh, bh,
                   wq, bq, wk, bk, wv, bv, wo, bo, lg, lb, o_ref):
    def bn(y, g, t):
        mu = jnp.mean(y, axis=0, keepdims=True)
        yc = y - mu
        var = jnp.mean(yc * yc, axis=0, keepdims=True)
        return jnp.maximum(g[...] * yc * jax.lax.rsqrt(var + 1e-5) + t[...], 0.0)

    aux = aux_ref[...]
    h = bn(jnp.dot(aux, w1[...], preferred_element_type=jnp.float32) + b1[...], g1, t1)
    h = bn(jnp.dot(h, w2[...], preferred_element_type=jnp.float32) + b2[...], g2, t2)
    tok = jnp.dot(h, wh[...], preferred_element_type=jnp.float32) + bh[...]

    x = x_ref[...]
    d = x.shape[1]
    q = jnp.dot(x, wq[...], preferred_element_type=jnp.float32) + bq[...]
    k = jnp.dot(tok, wk[...], preferred_element_type=jnp.float32) + bk[...]
    v = jnp.dot(tok, wv[...], preferred_element_type=jnp.float32) + bv[...]
    gate = jax.nn.sigmoid(jnp.sum(q * k, axis=1, keepdims=True) * (d ** -0.5))
    h2 = x + jnp.dot(v * gate, wo[...], preferred_element_type=jnp.float32) + bo[...]
    mu = jnp.mean(h2, axis=1, keepdims=True)
    hc = h2 - mu
    var = jnp.mean(hc * hc, axis=1, keepdims=True)
    o_ref[...] = lg[...] * hc * jax.lax.rsqrt(var + 1e-5) + lb[...]


def _auxcross(params, stage, x, aux):
    b, n, d = x.shape
    t0, t1p = params["aux_trunk"]
    hd = params["aux_heads"][stage]
    c = params["cross"][stage]
    args = [x.reshape(b * n, d), aux.reshape(b * n, aux.shape[-1]),
            t0["W"], _r2(t0["b"]), _r2(t0["g"]), _r2(t0["beta"]),
            t1p["W"], _r2(t1p["b"]), _r2(t1p["g"]), _r2(t1p["beta"]),
            hd["W"], _r2(hd["b"]),
            c["wq"]["W"], _r2(c["wq"]["b"]), c["wk"]["W"], _r2(c["wk"]["b"]),
            c["wv"]["W"], _r2(c["wv"]["b"]), c["out"]["W"], _r2(c["out"]["b"]),
            _r2(c["ln_g"]), _r2(c["ln_b"])]
    out = pl.pallas_call(
        _auxcross_body,
        out_shape=jax.ShapeDtypeStruct((b * n, d), jnp.float32),
    )(*args)
    return out.reshape(b, n, d)


def _fps_body(px_ref, py_ref, pz_ref, cx_ref, cy_ref, cz_ref, *, m):
    px = px_ref[...]
    py = py_ref[...]
    pz = pz_ref[...]
    bsz, n = px.shape
    d = ((px - px[:, 0:1]) ** 2 + (py - py[:, 0:1]) ** 2
         + (pz - pz[:, 0:1]) ** 2)
    iota = jax.lax.broadcasted_iota(jnp.int32, (bsz, n), 1)
    iota_m = jax.lax.broadcasted_iota(jnp.int32, (bsz, m), 1)
    sel0 = (iota_m == 0).astype(jnp.float32)
    cx = sel0 * px[:, 0:1]
    cy = sel0 * py[:, 0:1]
    cz = sel0 * pz[:, 0:1]

    def body(t, carry):
        d, cx, cy, cz = carry
        mx = jnp.max(d, axis=1, keepdims=True)
        cand = jnp.where(d == mx, iota, n)
        i = jnp.min(cand, axis=1, keepdims=True)
        oh = iota == i
        pxi = jnp.sum(jnp.where(oh, px, 0.0), axis=1, keepdims=True)
        pyi = jnp.sum(jnp.where(oh, py, 0.0), axis=1, keepdims=True)
        pzi = jnp.sum(jnp.where(oh, pz, 0.0), axis=1, keepdims=True)
        sel = (iota_m == t).astype(jnp.float32)
        cx = cx + sel * pxi
        cy = cy + sel * pyi
        cz = cz + sel * pzi
        dn = (px - pxi) ** 2 + (py - pyi) ** 2 + (pz - pzi) ** 2
        return jnp.minimum(d, dn), cx, cy, cz

    _, cx, cy, cz = jax.lax.fori_loop(1, m, body, (d, cx, cy, cz))
    cx_ref[...] = cx
    cy_ref[...] = cy
    cz_ref[...] = cz


def _fps(pos, m):
    bsz, n, _ = pos.shape
    sds = jax.ShapeDtypeStruct((bsz, m), jnp.float32)
    cx, cy, cz = pl.pallas_call(
        functools.partial(_fps_body, m=m),
        out_shape=[sds, sds, sds],
    )(pos[:, :, 0], pos[:, :, 1], pos[:, :, 2])
    return jnp.stack([cx, cy, cz], axis=-1)


def _extract_nearest(d2, iota_c, n):
    mn = jnp.min(d2, axis=1, keepdims=True)
    cand = jnp.where(d2 == mn, iota_c, n)
    j = jnp.min(cand, axis=1, keepdims=True)
    ohb = iota_c == j
    return ohb, jnp.where(ohb, 1e30, d2)


def _split_hi_lo(s):
    n, w = s.shape
    wp = -(-w // 128) * 128
    if wp > w:
        s = jnp.concatenate([s, jnp.zeros((n, wp - w), jnp.float32)], axis=1)
    hi = s.astype(jnp.bfloat16).astype(jnp.float32)
    return jnp.concatenate([hi, s - hi], axis=1), wp


def _onehot_gather(ohb, s2, wp):
    g2 = jnp.dot(ohb.astype(jnp.float32), s2,
                 preferred_element_type=jnp.float32)
    return g2[:, :wp] + g2[:, wp:]


def _pairwise_d2(a, bT):
    d2 = None
    for c in range(3):
        dd = a[:, c:c + 1] - bT[c:c + 1, :]
        d2 = dd * dd if d2 is None else d2 + dd * dd
    return d2


def _trans_body(xtr_ref, aux_ref, ctr_ref, posT_ref, xo_ref, ao_ref, *, k):
    xtr = xtr_ref[0]
    aux = aux_ref[0]
    ctr = ctr_ref[0]
    posT = posT_ref[0]
    n, dout = xtr.shape
    m = ctr.shape[0]
    d2 = _pairwise_d2(ctr, posT)
    iota_c = jax.lax.broadcasted_iota(jnp.int32, (m, n), 1)
    s2, wp = _split_hi_lo(jnp.concatenate([xtr, aux], axis=1))
    xmax = None
    asum = None
    for t in range(k):
        ohb, d2 = _extract_nearest(d2, iota_c, n)
        g = _onehot_gather(ohb, s2, wp)
        gx = g[:, :dout]
        ga = g[:, dout:dout + 4]
        xmax = gx if xmax is None else jnp.maximum(xmax, gx)
        asum = ga if asum is None else asum + ga
    xo_ref[0] = xmax
    ao_ref[0] = asum / k


def _trans(xtr, aux, pos, centers, k):
    b, n, dout = xtr.shape
    m = centers.shape[1]
    posT = jnp.swapaxes(pos, 1, 2)

    def bat(*shape):
        return pl.BlockSpec((1,) + shape, lambda i, s=len(shape): (i,) + (0,) * s)

    xo, ao = pl.pallas_call(
        functools.partial(_trans_body, k=k),
        grid=(b,),
        compiler_params=pltpu.CompilerParams(dimension_semantics=("parallel",)),
        in_specs=[bat(n, dout), bat(n, 4), bat(m, 3), bat(3, n)],
        out_specs=[bat(m, dout), bat(m, 4)],
        out_shape=[jax.ShapeDtypeStruct((b, m, dout), jnp.float32),
                   jax.ShapeDtypeStruct((b, m, 4), jnp.float32)],
    )(xtr, aux, centers, posT)
    return xo, ao


def _tblock_body(x_ref, pos_ref, posT_ref,
                 wi, bi, wl, ws, wd, wp1, bp1, wp2, bp2,
                 wa1, ba1, wa2, ba2, wo, bo, o_ref, *, k):
    def lin_relu(a, w, bb):
        return jnp.maximum(
            jnp.dot(a, w[...], preferred_element_type=jnp.float32) + bb[...], 0.0)

    x = x_ref[0]
    pos = pos_ref[0]
    posT = posT_ref[0]
    n, d = x.shape
    x1 = lin_relu(x, wi, bi)
    xl = jnp.dot(x1, wl[...], preferred_element_type=jnp.float32)
    asrc = jnp.dot(x1, ws[...], preferred_element_type=jnp.float32)
    adst = jnp.dot(x1, wd[...], preferred_element_type=jnp.float32)

    d2 = _pairwise_d2(pos, posT)
    iota_r = jax.lax.broadcasted_iota(jnp.int32, (n, n), 0)
    iota_c = jax.lax.broadcasted_iota(jnp.int32, (n, n), 1)
    d2 = d2 + jnp.where(iota_r == iota_c, 1e10, 0.0)

    s2, wp = _split_hi_lo(
        jnp.concatenate([pos, asrc, xl], axis=1))
    chunk = 1 if n % 8 else max(1, min(k, 4096 // n))
    es = []
    vs = []
    for t0 in range(0, k, chunk):
        pds, als, xjs = [], [], []
        for t in range(t0, min(t0 + chunk, k)):
            ohb, d2 = _extract_nearest(d2, iota_c, n)
            g = _onehot_gather(ohb, s2, wp)
            pds.append(pos - g[:, :3])
            als.append(adst - g[:, 3:3 + d])
            xjs.append(g[:, 3 + d:3 + 2 * d])
        c = len(pds)
        pd_c = pds[0] if c == 1 else jnp.concatenate(pds, axis=0)
        al_c = als[0] if c == 1 else jnp.concatenate(als, axis=0)
        delta_c = lin_relu(lin_relu(pd_c, wp1, bp1), wp2, bp2)
        e_c = lin_relu(lin_relu(al_c + delta_c, wa1, ba1), wa2, ba2)
        for ci in range(c):
            sl = slice(ci * n, (ci + 1) * n)
            es.append(e_c[sl])
            vs.append(xjs[ci] + delta_c[sl])
    mx = functools.reduce(jnp.maximum, es)
    ssum = None
    acc = None
    for e, v in zip(es, vs):
        w = jnp.exp(e - mx)
        ssum = w if ssum is None else ssum + w
        acc = w * v if acc is None else acc + w * v
    o_ref[0] = lin_relu(acc / ssum, wo, bo)


def _tblock(p, x, pos, k):
    b, n, d = x.shape
    posT = jnp.swapaxes(pos, 1, 2)
    pars = [p["lin_in"]["W"], _r2(p["lin_in"]["b"]), p["lin"]["W"],
            p["lin_src"]["W"], p["lin_dst"]["W"],
            p["pos_nn"][0]["W"], _r2(p["pos_nn"][0]["b"]),
            p["pos_nn"][1]["W"], _r2(p["pos_nn"][1]["b"]),
            p["attn_nn"][0]["W"], _r2(p["attn_nn"][0]["b"]),
            p["attn_nn"][1]["W"], _r2(p["attn_nn"][1]["b"]),
            p["lin_out"]["W"], _r2(p["lin_out"]["b"])]

    def bat(*shape):
        return pl.BlockSpec((1,) + shape, lambda i, s=len(shape): (i,) + (0,) * s)

    def shared(a):
        return pl.BlockSpec(a.shape, lambda i, s=a.ndim: (0,) * s)

    out = pl.pallas_call(
        functools.partial(_tblock_body, k=k),
        grid=(b,),
        compiler_params=pltpu.CompilerParams(dimension_semantics=("parallel",)),
        in_specs=[bat(n, d), bat(n, 3), bat(3, n)] + [shared(a) for a in pars],
        out_specs=bat(n, d),
        out_shape=jax.ShapeDtypeStruct((b, n, d), jnp.float32),
    )(x, pos, posT, *pars)
    return out


def _head_body(x_ref, ac_ref, w1, b1, w2, b2, o_ref, *, bsz, n):
    pooled = []
    for b in range(bsz):
        xb = x_ref[pl.ds(b * n, n), :]
        w = jax.nn.sigmoid(ac_ref[b:b + 1, :])
        s = jnp.dot(w, xb, precision=jax.lax.Precision.HIGHEST,
                    preferred_element_type=jnp.float32)
        ws = jnp.clip(jnp.sum(w, axis=1, keepdims=True), 1e-6)
        pooled.append(s / ws)
    pooled = jnp.concatenate(pooled, axis=0)
    h = jnp.maximum(
        jnp.dot(pooled, w1[...], preferred_element_type=jnp.float32) + b1[...], 0.0)
    o_ref[...] = jnp.dot(h, w2[...], preferred_element_type=jnp.float32) + b2[...]


def _head(p, x, aux):
    b, n, d = x.shape
    return pl.pallas_call(
        functools.partial(_head_body, bsz=b, n=n),
        out_shape=jax.ShapeDtypeStruct((b, p[1]["W"].shape[1]), jnp.float32),
    )(x.reshape(b * n, d), aux[:, :, 2], p[0]["W"], _r2(p[0]["b"]),
      p[1]["W"], _r2(p[1]["b"]))


def kernel(data, params):
    pos = data[..., :3]
    aux = data[..., 3:7]
    n = pos.shape[1]

    x = _lin_bn_relu(params["mlp_input"], pos)
    x = _tblock(params["tr_in"], x, pos, min(16, n))
    x = _auxcross(params, 0, x, aux)
    for i in range(4):
        m = int(np.ceil(n * 0.25))
        centers = _fps(pos, m)
        xtr = _lin_bn_relu(params["td"][i], x)
        x, aux = _trans(xtr, aux, pos, centers, min(16, n))
        pos = centers
        n = m
        x = _tblock(params["tr_down"][i], x, pos, min(16, n))
        x = _auxcross(params, i + 1, x, aux)
    return _head(params["head"], x, aux)
